# Initial kernel scaffold; baseline (speedup 1.0000x reference)
import jax
import jax.numpy as jnp
from jax.experimental import pallas as pl


def kernel(x, map_table):
    raise NotImplementedError("write your pallas kernel here")


# speedup vs baseline: 33.9688x; 33.9688x over previous
import functools

import jax
import jax.numpy as jnp
from jax import lax
from jax.experimental import pallas as pl
from jax.experimental.pallas import tpu as pltpu
from jax.experimental.pallas import tpu_sc as plsc

_B = 16384
_L = 200
_C = 5
_VPAD = 48
_NW = 32
_ROWS_W = _B // _NW
_CHUNK_ROWS = 32
_NCHUNK = _ROWS_W // _CHUNK_ROWS
_XCH = _CHUNK_ROWS * _L
_OCH = _CHUNK_ROWS * _C * _L
_PAIRS = _CHUNK_ROWS // 2
_VECS = 2 * _L // 16

_mesh = plsc.VectorSubcoreMesh(core_axis_name="c", subcore_axis_name="s")


@functools.partial(
    pl.kernel,
    out_type=jax.ShapeDtypeStruct((_B * _C * _L,), jnp.float32),
    mesh=_mesh,
    compiler_params=pltpu.CompilerParams(needs_layout_passes=False),
    scratch_types=[
        pltpu.VMEM((_VPAD,), jnp.float32),
        pltpu.VMEM((_VPAD,), jnp.float32),
        pltpu.VMEM((_VPAD,), jnp.float32),
        pltpu.VMEM((_VPAD,), jnp.float32),
        pltpu.VMEM((_VPAD,), jnp.float32),
        pltpu.VMEM((_XCH,), jnp.int32),
        pltpu.VMEM((_XCH,), jnp.int32),
        pltpu.VMEM((_OCH,), jnp.float32),
        pltpu.VMEM((_OCH,), jnp.float32),
        pltpu.SemaphoreType.DMA,
        pltpu.SemaphoreType.DMA,
        pltpu.SemaphoreType.DMA,
        pltpu.SemaphoreType.DMA,
    ],
)
def _lookup_kernel(xf, tabf, outf, t0, t1, t2, t3, t4, x_a, x_b, o_a, o_b,
                   sx_a, sx_b, so_a, so_b):
    cid = lax.axis_index("c")
    sid = lax.axis_index("s")
    wid = sid * 2 + cid

    tabs = (t0, t1, t2, t3, t4)
    for c in range(_C):
        pltpu.sync_copy(tabf.at[pl.ds(c * _VPAD, _VPAD)], tabs[c])

    iota = lax.iota(jnp.int32, 16)
    mid = iota + jnp.where(iota >= 8, 192 + 800, 192)

    xbase0 = wid * (_ROWS_W * _L)
    obase0 = wid * (_ROWS_W * _C * _L)

    def compute(x_v, o_v):
        def pair_body(p, carry):
            xp = p * (2 * _L)
            op = p * (2 * _C * _L)
            for j in range(_VECS):
                xv = x_v[pl.ds(xp + 16 * j, 16)]
                for c in range(_C):
                    vals = plsc.load_gather(tabs[c], [xv])
                    if j < 12:
                        o_v[pl.ds(op + c * _L + 16 * j, 16)] = vals
                    elif j == 12:
                        plsc.store_scatter(o_v, [mid + (op + c * _L)], vals)
                    else:
                        o_v[pl.ds(op + 800 + c * _L + 16 * j, 16)] = vals
            return carry
        lax.fori_loop(0, _PAIRS, pair_body, 0)

    xbufs = (x_a, x_b)
    obufs = (o_a, o_b)
    xsems = (sx_a, sx_b)
    osems = (so_a, so_b)

    xwait = [None, None]
    owait = [None, None]
    for b in range(2):
        xwait[b] = pltpu.async_copy(
            xf.at[pl.ds(xbase0 + b * _XCH, _XCH)], xbufs[b], xsems[b])

    for k in range(_NCHUNK):
        b = k % 2
        xwait[b].wait()
        if owait[b] is not None:
            owait[b].wait()
        compute(xbufs[b], obufs[b])
        owait[b] = pltpu.async_copy(
            obufs[b], outf.at[pl.ds(obase0 + k * _OCH, _OCH)], osems[b])
        if k + 2 < _NCHUNK:
            xwait[b] = pltpu.async_copy(
                xf.at[pl.ds(xbase0 + (k + 2) * _XCH, _XCH)], xbufs[b], xsems[b])

    for b in range(2):
        owait[b].wait()


def kernel(x, map_table):
    b, l = x.shape
    tab = jnp.zeros((_C, _VPAD), jnp.float32).at[:, : map_table.shape[0]].set(
        map_table.T
    )
    outf = _lookup_kernel(x.reshape(-1), tab.reshape(-1))
    return outf.reshape(b, _C, l)


# speedup vs baseline: 44.1096x; 1.2985x over previous
import functools

import jax
import jax.numpy as jnp
from jax import lax
from jax.experimental import pallas as pl
from jax.experimental.pallas import tpu as pltpu
from jax.experimental.pallas import tpu_sc as plsc

_B = 16384
_L = 200
_C = 5
_VPAD = 48
_NW = 32
_ROWS_W = _B // _NW
_CHUNK_ROWS = 32
_NCHUNK = _ROWS_W // _CHUNK_ROWS
_XCH = _CHUNK_ROWS * _L
_OCH = _CHUNK_ROWS * _C * _L
_PAIRS = _CHUNK_ROWS // 2
_VECS = 2 * _L // 16

_mesh = plsc.VectorSubcoreMesh(core_axis_name="c", subcore_axis_name="s")


@functools.partial(
    pl.kernel,
    out_type=jax.ShapeDtypeStruct((_B * _C * _L,), jnp.float32),
    mesh=_mesh,
    compiler_params=pltpu.CompilerParams(needs_layout_passes=False),
    scratch_types=[
        pltpu.VMEM((_VPAD,), jnp.float32),
        pltpu.VMEM((_VPAD,), jnp.float32),
        pltpu.VMEM((_VPAD,), jnp.float32),
        pltpu.VMEM((_VPAD,), jnp.float32),
        pltpu.VMEM((_VPAD,), jnp.float32),
        pltpu.VMEM((_XCH,), jnp.int32),
        pltpu.VMEM((_XCH,), jnp.int32),
        pltpu.VMEM((_OCH,), jnp.float32),
        pltpu.VMEM((_OCH,), jnp.float32),
        pltpu.SemaphoreType.DMA,
        pltpu.SemaphoreType.DMA,
        pltpu.SemaphoreType.DMA,
        pltpu.SemaphoreType.DMA,
    ],
)
def _lookup_kernel(xf, tabf, outf, t0, t1, t2, t3, t4, x_a, x_b, o_a, o_b,
                   sx_a, sx_b, so_a, so_b):
    cid = lax.axis_index("c")
    sid = lax.axis_index("s")
    wid = sid * 2 + cid

    tabs = (t0, t1, t2, t3, t4)
    for c in range(_C):
        pltpu.sync_copy(tabf.at[pl.ds(c * _VPAD, _VPAD)], tabs[c])

    iota = lax.iota(jnp.int32, 16)
    mid = iota + jnp.where(iota >= 8, 192 + 800, 192)

    xbase0 = wid * (_ROWS_W * _L)
    obase0 = wid * (_ROWS_W * _C * _L)

    def compute(x_v, o_v):
        def emit_stores(o_v, op, j, vals):
            for c in range(_C):
                if j < 12:
                    o_v[pl.ds(op + c * _L + 16 * j, 16)] = vals[c]
                elif j == 12:
                    plsc.store_scatter(o_v, [mid + (op + c * _L)], vals[c])
                else:
                    o_v[pl.ds(op + 800 + c * _L + 16 * j, 16)] = vals[c]

        def pair_body(p, carry):
            xp = p * (2 * _L)
            op = p * (2 * _C * _L)
            pend = None
            for j in range(_VECS):
                xv = x_v[pl.ds(xp + 16 * j, 16)]
                vals = [plsc.load_gather(tabs[c], [xv]) for c in range(_C)]
                if pend is not None:
                    emit_stores(o_v, op, j - 1, pend)
                pend = vals
            emit_stores(o_v, op, _VECS - 1, pend)
            return carry
        lax.fori_loop(0, _PAIRS, pair_body, 0)

    xbufs = (x_a, x_b)
    obufs = (o_a, o_b)
    xsems = (sx_a, sx_b)
    osems = (so_a, so_b)

    xwait = [None, None]
    owait = [None, None]
    for b in range(2):
        xwait[b] = pltpu.async_copy(
            xf.at[pl.ds(xbase0 + b * _XCH, _XCH)], xbufs[b], xsems[b])

    for k in range(_NCHUNK):
        b = k % 2
        xwait[b].wait()
        if owait[b] is not None:
            owait[b].wait()
        compute(xbufs[b], obufs[b])
        owait[b] = pltpu.async_copy(
            obufs[b], outf.at[pl.ds(obase0 + k * _OCH, _OCH)], osems[b])
        if k + 2 < _NCHUNK:
            xwait[b] = pltpu.async_copy(
                xf.at[pl.ds(xbase0 + (k + 2) * _XCH, _XCH)], xbufs[b], xsems[b])

    for b in range(2):
        owait[b].wait()


def kernel(x, map_table):
    b, l = x.shape
    tab = jnp.zeros((_C, _VPAD), jnp.float32).at[:, : map_table.shape[0]].set(
        map_table.T
    )
    outf = _lookup_kernel(x.reshape(-1), tab.reshape(-1))
    return outf.reshape(b, _C, l)


# speedup vs baseline: 64.0110x; 1.4512x over previous
import functools

import jax
import jax.numpy as jnp
from jax import lax
from jax.experimental import pallas as pl
from jax.experimental.pallas import tpu as pltpu
from jax.experimental.pallas import tpu_sc as plsc

_B = 16384
_L = 200
_C = 5
_VPAD = 48
_NW = 32
_ROWS_W = _B // _NW
_CHUNK_ROWS = 16
_NCHUNK = _ROWS_W // _CHUNK_ROWS
_PAIRS = _CHUNK_ROWS // 2
_VECS = 2 * _L // 16

_mesh = plsc.VectorSubcoreMesh(core_axis_name="c", subcore_axis_name="s")


@functools.partial(
    pl.kernel,
    out_type=jax.ShapeDtypeStruct((_B, _C, _L), jnp.float32),
    mesh=_mesh,
    compiler_params=pltpu.CompilerParams(needs_layout_passes=False),
    scratch_types=[
        pltpu.VMEM((_VPAD,), jnp.float32),
        pltpu.VMEM((_VPAD,), jnp.float32),
        pltpu.VMEM((_VPAD,), jnp.float32),
        pltpu.VMEM((_VPAD,), jnp.float32),
        pltpu.VMEM((_VPAD,), jnp.float32),
        pltpu.VMEM((_CHUNK_ROWS, _L), jnp.int32),
        pltpu.VMEM((_CHUNK_ROWS, _L), jnp.int32),
        pltpu.VMEM((_CHUNK_ROWS, _C, _L), jnp.float32),
        pltpu.VMEM((_CHUNK_ROWS, _C, _L), jnp.float32),
        pltpu.SemaphoreType.DMA,
        pltpu.SemaphoreType.DMA,
        pltpu.SemaphoreType.DMA,
        pltpu.SemaphoreType.DMA,
    ],
)
def _lookup_kernel(xf, tabf, outf, t0, t1, t2, t3, t4, x_a, x_b, o_a, o_b,
                   sx_a, sx_b, so_a, so_b):
    cid = lax.axis_index("c")
    sid = lax.axis_index("s")
    wid = sid * 2 + cid

    tabs = (t0, t1, t2, t3, t4)
    for c in range(_C):
        pltpu.sync_copy(tabf.at[pl.ds(c * _VPAD, _VPAD)], tabs[c])

    iota = lax.iota(jnp.int32, 16)
    rowadd = jnp.where(iota >= 8, 1, 0)
    colmid = jnp.where(iota >= 8, iota - 8, iota + 192)
    chans = [jnp.full((16,), c, jnp.int32) for c in range(_C)]

    row0w = wid * _ROWS_W

    def compute(x_v, o_v):
        def emit_stores(op, j, vals, rvec):
            for c in range(_C):
                if j < 12:
                    o_v[2 * op, c, pl.ds(16 * j, 16)] = vals[c]
                elif j == 12:
                    plsc.store_scatter(o_v, [rvec, chans[c], colmid], vals[c])
                else:
                    o_v[2 * op + 1, c, pl.ds(16 * j - _L, 16)] = vals[c]

        def pair_body(p, carry):
            rvec = rowadd + 2 * p
            pend = None
            for j in range(_VECS):
                if j < 12:
                    xv = x_v[2 * p, pl.ds(16 * j, 16)]
                elif j == 12:
                    xv = plsc.load_gather(x_v, [rvec, colmid])
                else:
                    xv = x_v[2 * p + 1, pl.ds(16 * j - _L, 16)]
                vals = [plsc.load_gather(tabs[c], [xv]) for c in range(_C)]
                if pend is not None:
                    emit_stores(p, j - 1, pend, rvec)
                pend = vals
            emit_stores(p, _VECS - 1, pend, rvec)
            return carry
        lax.fori_loop(0, _PAIRS, pair_body, 0)

    xbufs = (x_a, x_b)
    obufs = (o_a, o_b)
    xsems = (sx_a, sx_b)
    osems = (so_a, so_b)

    for b in range(2):
        pltpu.async_copy(
            xf.at[pl.ds(row0w + b * _CHUNK_ROWS, _CHUNK_ROWS)],
            xbufs[b], xsems[b])

    def ring_body(k2, carry):
        for b in range(2):
            k = 2 * k2 + b
            row0 = row0w + k * _CHUNK_ROWS
            pltpu.make_async_copy(
                xf.at[pl.ds(row0, _CHUNK_ROWS)], xbufs[b], xsems[b]).wait()

            @pl.when(k2 > 0)
            def _():
                pltpu.make_async_copy(
                    obufs[b], outf.at[pl.ds(row0w, _CHUNK_ROWS)],
                    osems[b]).wait()

            compute(xbufs[b], obufs[b])
            pltpu.async_copy(
                obufs[b], outf.at[pl.ds(row0, _CHUNK_ROWS)], osems[b])

            @pl.when(k2 < _NCHUNK // 2 - 1)
            def _():
                pltpu.async_copy(
                    xf.at[pl.ds(row0 + 2 * _CHUNK_ROWS, _CHUNK_ROWS)],
                    xbufs[b], xsems[b])
        return carry

    lax.fori_loop(0, _NCHUNK // 2, ring_body, 0)

    for b in range(2):
        pltpu.make_async_copy(
            obufs[b], outf.at[pl.ds(row0w, _CHUNK_ROWS)], osems[b]).wait()


def kernel(x, map_table):
    tab = jnp.zeros((_C, _VPAD), jnp.float32).at[:, : map_table.shape[0]].set(
        map_table.T
    )
    return _lookup_kernel(x, tab.reshape(-1))
